# baseline (speedup 1.0000x reference)
import jax
import jax.numpy as jnp
from jax import lax
from jax.experimental import pallas as pl
from jax.experimental.pallas import tpu as pltpu
from jax.experimental.pallas import tpu_sc as plsc

_L = 16
_ROWS, _N = 128, 32768
_SLAB = 4096
_NSLABS = _N // _SLAB
_RING = 3
_CAPL = 1024
_CAPL2 = 256
_UN = 8
_UN2 = 4
_NEWTON_ITERS = 12
_NEG = -3.0e38


def _sc_body(x_hbm, o_hbm, s0, s1, s2, zbuf, cvals, cpos, c2vals, c2pos,
             sem_in, sem_out):
    info = plsc.get_sparse_core_info()
    nc, ns = info.num_cores, info.num_subcores
    rpw = _ROWS // (nc * ns)
    wid = lax.axis_index("s") * nc + lax.axis_index("c")
    lane = lax.iota(jnp.int32, _L)
    base = lane * _CAPL
    base2 = lane * _CAPL2
    ones = jnp.ones((_L,), jnp.int32)
    zero = jnp.zeros((_L,), jnp.float32)
    izero = jnp.zeros((_L,), jnp.int32)
    sent = jnp.full((_L,), _NEG, jnp.float32)
    bufs = [s0, s1, s2]

    def z_body(i, _):
        for u in range(_UN):
            zbuf[pl.ds((i * _UN + u) * _L, _L)] = zero
        return 0
    lax.fori_loop(0, _N // _L // _UN, z_body, 0)

    hout = None
    prev = None

    for r in range(rpw):
        row = wid * rpw + r
        rowref = x_hbm.at[row]

        hin = {}
        for s in range(_RING - 1):
            hin[s] = pltpu.async_copy(
                rowref.at[pl.ds(s * _SLAB, _SLAB)], bufs[s % _RING], sem_in)

        carry = (jnp.full((_L,), _NEG, jnp.float32),
                 jnp.zeros((_L,), jnp.int32),
                 lane)
        for s in range(_NSLABS):
            hin[s].wait()
            nxt = s + _RING - 1
            if nxt < _NSLABS:
                hin[nxt] = pltpu.async_copy(
                    rowref.at[pl.ds(nxt * _SLAB, _SLAB)], bufs[nxt % _RING],
                    sem_in)
            buf = bufs[s % _RING]

            def scan_body(i, c, buf=buf):
                rm1, off, pos = c
                for u in range(_UN):
                    v = buf[pl.ds((i * _UN + u) * _L, _L)]
                    keep = v > rm1
                    idx = base + jnp.minimum(off, _CAPL - 1)
                    plsc.store_scatter(cvals, [idx], v, mask=keep)
                    plsc.store_scatter(cpos, [idx], pos, mask=keep)
                    rm1 = jnp.maximum(rm1, v - 1.0)
                    off = off + jnp.where(keep, ones, 0)
                    pos = pos + _L
                return rm1, off, pos

            carry = lax.fori_loop(0, _SLAB // _L // _UN, scan_body, carry)

        rm1, cnt, _ = carry
        m = jnp.max(rm1) + 1.0
        thr = jnp.broadcast_to(m - 1.0, (_L,))
        nch = jnp.max(cnt)

        if hout is not None:
            hout.wait()
            pn, _ptau = prev

            def rst_body(i, _):
                for u in range(_UN2):
                    p = plsc.load_gather(c2pos, [base2 + (i * _UN2 + u)])
                    plsc.store_scatter(zbuf, [p], zero)
                return 0
            lax.fori_loop(0, pn, rst_body, 0)

        def rec_body(i, off2):
            for u in range(_UN2):
                j = i * _UN2 + u
                v = plsc.load_gather(cvals, [base + j])
                p = plsc.load_gather(cpos, [base + j])
                valid = (j < cnt) & (v > thr)
                idx2 = base2 + jnp.minimum(off2, _CAPL2 - 1)
                plsc.store_scatter(c2vals, [idx2], v, mask=valid)
                plsc.store_scatter(c2pos, [idx2], p, mask=valid)
                off2 = off2 + jnp.where(valid, ones, 0)
            return off2

        n_rec = (nch + (_UN2 - 1)) // _UN2
        cnt2 = lax.fori_loop(0, n_rec, rec_body, jnp.zeros((_L,), jnp.int32))

        nch2 = jnp.max(cnt2)
        n_new = (nch2 + (_UN2 - 1)) // _UN2

        def fill_body(j, _):
            idxf = base2 + jnp.minimum(j, _CAPL2 - 1)
            plsc.store_scatter(c2vals, [idxf], sent, mask=j >= cnt2)
            plsc.store_scatter(c2pos, [idxf], izero, mask=j >= cnt2)
            return 0
        lax.fori_loop(0, n_new * _UN2, fill_body, 0)

        def newton(_, t):
            def ch(i, acc):
                sacc, kacc = acc
                for u in range(_UN2):
                    v = plsc.load_gather(c2vals, [base2 + (i * _UN2 + u)])
                    act = v > t
                    sacc = sacc + jnp.where(act, v, 0.0)
                    kacc = kacc + jnp.where(act, 1.0, 0.0)
                return sacc, kacc
            sacc, kacc = lax.fori_loop(0, n_new, ch, (zero, zero))
            sv = jnp.broadcast_to(jnp.sum(sacc) - 1.0, (_L,))
            kv = jnp.broadcast_to(jnp.sum(kacc), (_L,))
            return sv / kv

        tau = lax.fori_loop(0, _NEWTON_ITERS, newton,
                            jnp.broadcast_to(m - 1.0, (_L,)))

        def sc_out(i, _):
            for u in range(_UN2):
                j = i * _UN2 + u
                v = plsc.load_gather(c2vals, [base2 + j])
                p = plsc.load_gather(c2pos, [base2 + j])
                plsc.store_scatter(zbuf, [p], jnp.maximum(v - tau, 0.0))
            return 0
        lax.fori_loop(0, n_new, sc_out, 0)

        hout = pltpu.async_copy(zbuf, o_hbm.at[row], sem_out)
        prev = (n_new, tau)

    hout.wait()


@jax.jit
def kernel(input_tensor):
    mesh = plsc.VectorSubcoreMesh(core_axis_name="c", subcore_axis_name="s")
    return pl.kernel(
        _sc_body,
        out_type=jax.ShapeDtypeStruct((_ROWS, _N), jnp.float32),
        mesh=mesh,
        scratch_types=[
            pltpu.VMEM((_SLAB,), jnp.float32),
            pltpu.VMEM((_SLAB,), jnp.float32),
            pltpu.VMEM((_SLAB,), jnp.float32),
            pltpu.VMEM((_N,), jnp.float32),
            pltpu.VMEM((_L * _CAPL,), jnp.float32),
            pltpu.VMEM((_L * _CAPL,), jnp.int32),
            pltpu.VMEM((_L * _CAPL2,), jnp.float32),
            pltpu.VMEM((_L * _CAPL2,), jnp.int32),
            pltpu.SemaphoreType.DMA,
            pltpu.SemaphoreType.DMA,
        ],
        compiler_params=pltpu.CompilerParams(needs_layout_passes=False),
    )(input_tensor)


# speedup vs baseline: 1.0918x; 1.0918x over previous
import jax
import jax.numpy as jnp
from jax import lax
from jax.experimental import pallas as pl
from jax.experimental.pallas import tpu as pltpu
from jax.experimental.pallas import tpu_sc as plsc

_L = 16
_ROWS, _N = 128, 32768
_SLAB = 4096
_NSLABS = _N // _SLAB
_RING = 3
_CAPL = 1024
_CAPL2 = 256
_UN = 8
_UN2 = 4
_NEWTON_ITERS = 12
_NEG = -3.0e38


def _sc_body(x_hbm, o_hbm, s0, s1, s2, zbuf, cvals, cpos, c2vals, c2pos,
             sem_in, sem_out):
    info = plsc.get_sparse_core_info()
    nc, ns = info.num_cores, info.num_subcores
    rpw = _ROWS // (nc * ns)
    wid = lax.axis_index("s") * nc + lax.axis_index("c")
    lane = lax.iota(jnp.int32, _L)
    ones = jnp.ones((_L,), jnp.int32)
    zero = jnp.zeros((_L,), jnp.float32)
    izero = jnp.zeros((_L,), jnp.int32)
    sent = jnp.full((_L,), _NEG, jnp.float32)
    bufs = [s0, s1, s2]

    def z_body(i, _):
        for u in range(_UN):
            zbuf[pl.ds((i * _UN + u) * _L, _L)] = zero
        return 0
    lax.fori_loop(0, _N // _L // _UN, z_body, 0)

    hout = None
    prev = None

    for r in range(rpw):
        row = wid * rpw + r
        rowref = x_hbm.at[row]

        hin = {}
        for s in range(_RING - 1):
            hin[s] = pltpu.async_copy(
                rowref.at[pl.ds(s * _SLAB, _SLAB)], bufs[s % _RING], sem_in)

        carry = (jnp.full((_L,), _NEG, jnp.float32),
                 jnp.zeros((_L,), jnp.int32),
                 lane)
        for s in range(_NSLABS):
            hin[s].wait()
            nxt = s + _RING - 1
            if nxt < _NSLABS:
                hin[nxt] = pltpu.async_copy(
                    rowref.at[pl.ds(nxt * _SLAB, _SLAB)], bufs[nxt % _RING],
                    sem_in)
            buf = bufs[s % _RING]

            def scan_body(i, c, buf=buf):
                rm1, off, pos = c
                for u in range(_UN):
                    v = buf[pl.ds((i * _UN + u) * _L, _L)]
                    keep = v > rm1
                    idx = jnp.minimum(off, _CAPL - 1) * _L + lane
                    plsc.store_scatter(cvals, [idx], v, mask=keep)
                    plsc.store_scatter(cpos, [idx], pos, mask=keep)
                    rm1 = jnp.maximum(rm1, v - 1.0)
                    off = off + jnp.where(keep, ones, 0)
                    pos = pos + _L
                return rm1, off, pos

            carry = lax.fori_loop(0, _SLAB // _L // _UN, scan_body, carry)

        rm1, cnt, _ = carry
        m = jnp.max(rm1) + 1.0
        thr = jnp.broadcast_to(m - 1.0, (_L,))
        nch = jnp.max(cnt)

        if hout is not None:
            hout.wait()
            pn, _ptau = prev

            def rst_body(i, _):
                for u in range(_UN2):
                    p = plsc.load_gather(c2pos, [(i * _UN2 + u) * _L + lane])
                    plsc.store_scatter(zbuf, [p], zero)
                return 0
            lax.fori_loop(0, pn, rst_body, 0)

        def rec_body(i, off2):
            for u in range(_UN2):
                j = i * _UN2 + u
                v = plsc.load_gather(cvals, [j * _L + lane])
                p = plsc.load_gather(cpos, [j * _L + lane])
                valid = (j < cnt) & (v > thr)
                idx2 = jnp.minimum(off2, _CAPL2 - 1) * _L + lane
                plsc.store_scatter(c2vals, [idx2], v, mask=valid)
                plsc.store_scatter(c2pos, [idx2], p, mask=valid)
                off2 = off2 + jnp.where(valid, ones, 0)
            return off2

        n_rec = (nch + (_UN2 - 1)) // _UN2
        cnt2 = lax.fori_loop(0, n_rec, rec_body, jnp.zeros((_L,), jnp.int32))

        nch2 = jnp.max(cnt2)
        n_new = (nch2 + (_UN2 - 1)) // _UN2

        def fill_body(j, _):
            idxf = jnp.minimum(j, _CAPL2 - 1) * _L + lane
            plsc.store_scatter(c2vals, [idxf], sent, mask=j >= cnt2)
            plsc.store_scatter(c2pos, [idxf], izero, mask=j >= cnt2)
            return 0
        lax.fori_loop(0, n_new * _UN2, fill_body, 0)

        def newton(_, t):
            def ch(i, acc):
                sacc, kacc = acc
                for u in range(_UN2):
                    v = plsc.load_gather(c2vals, [(i * _UN2 + u) * _L + lane])
                    act = v > t
                    sacc = sacc + jnp.where(act, v, 0.0)
                    kacc = kacc + jnp.where(act, 1.0, 0.0)
                return sacc, kacc
            sacc, kacc = lax.fori_loop(0, n_new, ch, (zero, zero))
            sv = jnp.broadcast_to(jnp.sum(sacc) - 1.0, (_L,))
            kv = jnp.broadcast_to(jnp.sum(kacc), (_L,))
            return sv / kv

        tau = lax.fori_loop(0, _NEWTON_ITERS, newton,
                            jnp.broadcast_to(m - 1.0, (_L,)))

        def sc_out(i, _):
            for u in range(_UN2):
                j = i * _UN2 + u
                v = plsc.load_gather(c2vals, [j * _L + lane])
                p = plsc.load_gather(c2pos, [j * _L + lane])
                plsc.store_scatter(zbuf, [p], jnp.maximum(v - tau, 0.0))
            return 0
        lax.fori_loop(0, n_new, sc_out, 0)

        hout = pltpu.async_copy(zbuf, o_hbm.at[row], sem_out)
        prev = (n_new, tau)

    hout.wait()


@jax.jit
def kernel(input_tensor):
    mesh = plsc.VectorSubcoreMesh(core_axis_name="c", subcore_axis_name="s")
    return pl.kernel(
        _sc_body,
        out_type=jax.ShapeDtypeStruct((_ROWS, _N), jnp.float32),
        mesh=mesh,
        scratch_types=[
            pltpu.VMEM((_SLAB,), jnp.float32),
            pltpu.VMEM((_SLAB,), jnp.float32),
            pltpu.VMEM((_SLAB,), jnp.float32),
            pltpu.VMEM((_N,), jnp.float32),
            pltpu.VMEM((_L * _CAPL,), jnp.float32),
            pltpu.VMEM((_L * _CAPL,), jnp.int32),
            pltpu.VMEM((_L * _CAPL2,), jnp.float32),
            pltpu.VMEM((_L * _CAPL2,), jnp.int32),
            pltpu.SemaphoreType.DMA,
            pltpu.SemaphoreType.DMA,
        ],
        compiler_params=pltpu.CompilerParams(needs_layout_passes=False),
    )(input_tensor)


# speedup vs baseline: 1.2242x; 1.1212x over previous
import jax
import jax.numpy as jnp
from jax import lax
from jax.experimental import pallas as pl
from jax.experimental.pallas import tpu as pltpu
from jax.experimental.pallas import tpu_sc as plsc

_L = 16
_ROWS, _N = 128, 32768
_UN = 8
_UN2 = 4
_CAPU = 128
_CAP2 = 256
_NEWTON_ITERS = 12
_NEG = -3.0e38


def _sc_body(x_hbm, o_hbm, rb0, rb1, zbuf, cpos, c2vals, c2pos,
             sem_in, sem_out):
    info = plsc.get_sparse_core_info()
    nc, ns = info.num_cores, info.num_subcores
    rpw = _ROWS // (nc * ns)
    wid = lax.axis_index("s") * nc + lax.axis_index("c")
    lane = lax.iota(jnp.int32, _L)
    zero = jnp.zeros((_L,), jnp.float32)
    izero = jnp.zeros((_L,), jnp.int32)
    sent = jnp.full((_L,), _NEG, jnp.float32)
    neg = jnp.full((_L,), _NEG, jnp.float32)
    rbufs = [rb0, rb1]
    row0 = wid * rpw
    _GROUPS = _N // _L // _UN

    hin = {0: pltpu.async_copy(x_hbm.at[row0], rbufs[0], sem_in)}

    def z_body(i, _):
        for u in range(_UN):
            zbuf[pl.ds((i * _UN + u) * _L, _L)] = zero
        return 0
    lax.fori_loop(0, _GROUPS, z_body, 0)

    def zc_body(i, _):
        for u in range(_UN):
            cpos[pl.ds((i * _UN + u) * _L, _L)] = izero
        return 0
    lax.fori_loop(0, _UN * _CAPU // _UN, zc_body, 0)

    hout = None
    prev_n = None

    for r in range(rpw):
        row = row0 + r
        hin[r].wait()
        if r + 1 < rpw:
            hin[r + 1] = pltpu.async_copy(
                x_hbm.at[row + 1], rbufs[(r + 1) % 2], sem_in)
        rbuf = rbufs[r % 2]

        def max_body(i, rms):
            return tuple(
                jnp.maximum(rms[u], rbuf[pl.ds((i * _UN + u) * _L, _L)])
                for u in range(_UN))

        rms = lax.fori_loop(0, _GROUPS, max_body, (neg,) * _UN)
        t = list(rms)
        while len(t) > 1:
            t = [jnp.maximum(t[i], t[i + 1]) for i in range(0, len(t), 2)]
        m = jnp.max(t[0])
        thr = jnp.broadcast_to(m - 1.0, (_L,))

        if hout is not None:
            hout.wait()

            def rst_body(i, _):
                for u in range(_UN2):
                    p = plsc.load_gather(c2pos, [(i * _UN2 + u) * _L + lane])
                    plsc.store_scatter(zbuf, [p], zero)
                return 0
            lax.fori_loop(0, prev_n, rst_body, 0)

        culane = [jnp.broadcast_to(u * _L, (_L,)) + lane for u in range(_UN)]
        i128 = jnp.full((_L,), _UN * _L, jnp.int32)

        def cand_body(i, c):
            offs = list(c[0])
            poss = list(c[1])
            for u in range(_UN):
                v = rbuf[pl.ds((i * _UN + u) * _L, _L)]
                keep = v > thr
                plsc.store_scatter(cpos, [offs[u]], poss[u], mask=keep)
                offs[u] = offs[u] + jnp.where(keep, i128, 0)
                poss[u] = poss[u] + (_UN * _L)
            return tuple(offs), tuple(poss)

        offs0 = tuple(culane)
        poss0 = tuple(jnp.broadcast_to(u * _L, (_L,)) + lane
                      for u in range(_UN))
        offs, _ = lax.fori_loop(0, _GROUPS, cand_body, (offs0, poss0))

        cnts = [lax.shift_right_logical(offs[u] - culane[u], 7)
                for u in range(_UN)]
        t = cnts
        while len(t) > 1:
            t = [jnp.maximum(t[i], t[i + 1]) for i in range(0, len(t), 2)]
        nch = jnp.max(t[0])

        def dense_body(j, c):
            off2, cnt_ = c
            for u in range(_UN):
                p = plsc.load_gather(cpos, [j * (_UN * _L) + culane[u]])
                v = plsc.load_gather(rbuf, [p])
                valid = (j < cnts[u]) & (v > thr)
                plsc.store_scatter(c2vals, [off2], v, mask=valid)
                plsc.store_scatter(c2pos, [off2], p, mask=valid)
                inc = jnp.where(valid, _L, 0)
                off2 = off2 + inc
                cnt_ = cnt_ + jnp.where(valid, 1, 0)
            return off2, cnt_

        off2, cnt2 = lax.fori_loop(0, nch, dense_body, (lane, izero))
        nch2 = jnp.max(cnt2)
        n_new = (nch2 + (_UN2 - 1)) // _UN2

        def fill_body(j, _):
            idxf = jnp.minimum(j, _CAP2 - 1) * _L + lane
            plsc.store_scatter(c2vals, [idxf], sent, mask=j >= cnt2)
            plsc.store_scatter(c2pos, [idxf], izero, mask=j >= cnt2)
            return 0
        lax.fori_loop(0, n_new * _UN2, fill_body, 0)

        def newton(_, t):
            def ch(i, acc):
                sacc, kacc = acc
                for u in range(_UN2):
                    v = plsc.load_gather(c2vals, [(i * _UN2 + u) * _L + lane])
                    act = v > t
                    sacc = sacc + jnp.where(act, v, 0.0)
                    kacc = kacc + jnp.where(act, 1.0, 0.0)
                return sacc, kacc
            sacc, kacc = lax.fori_loop(0, n_new, ch, (zero, zero))
            sv = jnp.broadcast_to(jnp.sum(sacc) - 1.0, (_L,))
            kv = jnp.broadcast_to(jnp.sum(kacc), (_L,))
            return sv / kv

        tau = lax.fori_loop(0, _NEWTON_ITERS, newton,
                            jnp.broadcast_to(m - 1.0, (_L,)))

        def sc_out(i, _):
            for u in range(_UN2):
                j = i * _UN2 + u
                v = plsc.load_gather(c2vals, [j * _L + lane])
                p = plsc.load_gather(c2pos, [j * _L + lane])
                plsc.store_scatter(zbuf, [p], jnp.maximum(v - tau, 0.0))
            return 0
        lax.fori_loop(0, n_new, sc_out, 0)

        hout = pltpu.async_copy(zbuf, o_hbm.at[row], sem_out)
        prev_n = n_new

    hout.wait()


@jax.jit
def kernel(input_tensor):
    mesh = plsc.VectorSubcoreMesh(core_axis_name="c", subcore_axis_name="s")
    return pl.kernel(
        _sc_body,
        out_type=jax.ShapeDtypeStruct((_ROWS, _N), jnp.float32),
        mesh=mesh,
        scratch_types=[
            pltpu.VMEM((_N,), jnp.float32),
            pltpu.VMEM((_N,), jnp.float32),
            pltpu.VMEM((_N,), jnp.float32),
            pltpu.VMEM((_UN * _L * _CAPU,), jnp.int32),
            pltpu.VMEM((_L * _CAP2,), jnp.float32),
            pltpu.VMEM((_L * _CAP2,), jnp.int32),
            pltpu.SemaphoreType.DMA,
            pltpu.SemaphoreType.DMA,
        ],
        compiler_params=pltpu.CompilerParams(needs_layout_passes=False),
    )(input_tensor)


# speedup vs baseline: 1.2316x; 1.0060x over previous
import jax
import jax.numpy as jnp
from jax import lax
from jax.experimental import pallas as pl
from jax.experimental.pallas import tpu as pltpu
from jax.experimental.pallas import tpu_sc as plsc

_L = 16
_ROWS, _N = 128, 32768
_UN = 8
_UN2 = 4
_CAPU = 128
_CAP2 = 256
_NEWTON_ITERS = 12
_NEG = -3.0e38


def _sc_body(x_hbm, o_hbm, rb0, rb1, zbuf, cpos, c2vals, c2pos,
             sem_in, sem_out):
    info = plsc.get_sparse_core_info()
    nc, ns = info.num_cores, info.num_subcores
    rpw = _ROWS // (nc * ns)
    wid = lax.axis_index("s") * nc + lax.axis_index("c")
    lane = lax.iota(jnp.int32, _L)
    zero = jnp.zeros((_L,), jnp.float32)
    izero = jnp.zeros((_L,), jnp.int32)
    sent = jnp.full((_L,), _NEG, jnp.float32)
    neg = jnp.full((_L,), _NEG, jnp.float32)
    rbufs = [rb0, rb1]
    row0 = wid * rpw
    _GROUPS = _N // _L // _UN

    hin = {0: pltpu.async_copy(x_hbm.at[row0], rbufs[0], sem_in)}

    def z_body(i, _):
        for u in range(_UN):
            zbuf[pl.ds((i * _UN + u) * _L, _L)] = zero
        return 0
    lax.fori_loop(0, _GROUPS, z_body, 0)

    def zc_body(i, _):
        for u in range(_UN):
            cpos[pl.ds((i * _UN + u) * _L, _L)] = izero
        return 0
    lax.fori_loop(0, _UN * _CAPU // _UN, zc_body, 0)

    hout = None
    prev_n = None

    for r in range(rpw):
        row = row0 + r
        hin[r].wait()
        if r + 1 < rpw:
            hin[r + 1] = pltpu.async_copy(
                x_hbm.at[row + 1], rbufs[(r + 1) % 2], sem_in)
        rbuf = rbufs[r % 2]

        def max_body(i, rms):
            return tuple(
                jnp.maximum(rms[u], rbuf[pl.ds((i * _UN + u) * _L, _L)])
                for u in range(_UN))

        rms = lax.fori_loop(0, _GROUPS, max_body, (neg,) * _UN)
        t = list(rms)
        while len(t) > 1:
            t = [jnp.maximum(t[i], t[i + 1]) for i in range(0, len(t), 2)]
        m = jnp.max(t[0])
        thr = jnp.broadcast_to(m - 1.0, (_L,))

        if hout is not None:
            hout.wait()

            def rst_body(i, _):
                for u in range(_UN2):
                    p = plsc.load_gather(c2pos, [(i * _UN2 + u) * _L + lane])
                    plsc.store_scatter(zbuf, [p], zero)
                return 0
            lax.fori_loop(0, prev_n, rst_body, 0)

        culane = [jnp.broadcast_to(u * _L, (_L,)) + lane for u in range(_UN)]
        i128 = jnp.full((_L,), _UN * _L, jnp.int32)

        def cand_body(i, c):
            offs = list(c[0])
            poss = list(c[1])
            for u in range(_UN):
                v = rbuf[pl.ds((i * _UN + u) * _L, _L)]
                keep = v > thr
                plsc.store_scatter(cpos, [offs[u]], poss[u], mask=keep)
                offs[u] = offs[u] + jnp.where(keep, i128, 0)
                poss[u] = poss[u] + (_UN * _L)
            return tuple(offs), tuple(poss)

        offs0 = tuple(culane)
        poss0 = tuple(jnp.broadcast_to(u * _L, (_L,)) + lane
                      for u in range(_UN))
        offs, _ = lax.fori_loop(0, _GROUPS, cand_body, (offs0, poss0))

        cnts = [lax.shift_right_logical(offs[u] - culane[u], 7)
                for u in range(_UN)]
        t = cnts
        while len(t) > 1:
            t = [jnp.maximum(t[i], t[i + 1]) for i in range(0, len(t), 2)]
        nch = jnp.max(t[0])

        def dense_body(j, c):
            off2, cnt_ = c
            for u in range(_UN):
                p = plsc.load_gather(cpos, [j * (_UN * _L) + culane[u]])
                v = plsc.load_gather(rbuf, [p])
                valid = (j < cnts[u]) & (v > thr)
                plsc.store_scatter(c2vals, [off2], v, mask=valid)
                plsc.store_scatter(c2pos, [off2], p, mask=valid)
                inc = jnp.where(valid, _L, 0)
                off2 = off2 + inc
                cnt_ = cnt_ + jnp.where(valid, 1, 0)
            return off2, cnt_

        off2, cnt2 = lax.fori_loop(0, nch, dense_body, (lane, izero))
        nch2 = jnp.max(cnt2)
        n_new = (nch2 + (_UN2 - 1)) // _UN2

        def fill_body(j, _):
            idxf = jnp.minimum(j, _CAP2 - 1) * _L + lane
            plsc.store_scatter(c2vals, [idxf], sent, mask=j >= cnt2)
            plsc.store_scatter(c2pos, [idxf], izero, mask=j >= cnt2)
            return 0
        lax.fori_loop(0, n_new * _UN2, fill_body, 0)

        def newton(_, t):
            def ch(i, acc):
                sacc, kacc = acc
                for u in range(_UN2):
                    v = plsc.load_gather(c2vals, [(i * _UN2 + u) * _L + lane])
                    act = v > t
                    sacc = sacc + jnp.where(act, v, 0.0)
                    kacc = kacc + jnp.where(act, 1.0, 0.0)
                return sacc, kacc
            sacc, kacc = lax.fori_loop(0, n_new, ch, (zero, zero))
            sv = jnp.broadcast_to(jnp.sum(sacc) - 1.0, (_L,))
            kv = jnp.broadcast_to(jnp.sum(kacc), (_L,))
            return sv / kv

        tau = lax.fori_loop(0, _NEWTON_ITERS, newton,
                            jnp.broadcast_to(m - 1.0, (_L,)))

        def sc_out(i, _):
            for u in range(_UN2):
                j = i * _UN2 + u
                v = plsc.load_gather(c2vals, [j * _L + lane])
                p = plsc.load_gather(c2pos, [j * _L + lane])
                plsc.store_scatter(zbuf, [p], jnp.maximum(v - tau, 0.0),
                                   mask=v > thr)
            return 0
        lax.fori_loop(0, n_new, sc_out, 0)

        hout = pltpu.async_copy(zbuf, o_hbm.at[row], sem_out)
        prev_n = n_new

    hout.wait()


@jax.jit
def kernel(input_tensor):
    mesh = plsc.VectorSubcoreMesh(core_axis_name="c", subcore_axis_name="s")
    return pl.kernel(
        _sc_body,
        out_type=jax.ShapeDtypeStruct((_ROWS, _N), jnp.float32),
        mesh=mesh,
        scratch_types=[
            pltpu.VMEM((_N,), jnp.float32),
            pltpu.VMEM((_N,), jnp.float32),
            pltpu.VMEM((_N,), jnp.float32),
            pltpu.VMEM((_UN * _L * _CAPU,), jnp.int32),
            pltpu.VMEM((_L * _CAP2,), jnp.float32),
            pltpu.VMEM((_L * _CAP2,), jnp.int32),
            pltpu.SemaphoreType.DMA,
            pltpu.SemaphoreType.DMA,
        ],
        compiler_params=pltpu.CompilerParams(needs_layout_passes=False),
    )(input_tensor)


# speedup vs baseline: 2.5293x; 2.0536x over previous
import jax
import jax.numpy as jnp
from jax import lax
from jax.experimental import pallas as pl
from jax.experimental.pallas import tpu as pltpu
from jax.experimental.pallas import tpu_sc as plsc

_L = 16
_ROWS, _N = 128, 32768
_UN = 8
_UN2 = 4
_GW = 16
_PV = 256
_CAP2 = 256
_NEWTON_ITERS = 12
_NEG = -3.0e38


def _sc_body(x_hbm, o_hbm, rb0, rb1, zbuf, gbuf, pairbuf, c2vals, c2pos,
             sem_in, sem_out):
    info = plsc.get_sparse_core_info()
    nc, ns = info.num_cores, info.num_subcores
    rpw = _ROWS // (nc * ns)
    wid = lax.axis_index("s") * nc + lax.axis_index("c")
    lane = lax.iota(jnp.int32, _L)
    zero = jnp.zeros((_L,), jnp.float32)
    izero = jnp.zeros((_L,), jnp.int32)
    sent = jnp.full((_L,), _NEG, jnp.float32)
    neg = jnp.full((_L,), _NEG, jnp.float32)
    rbufs = [rb0, rb1]
    row0 = wid * rpw
    _NG = _N // (_GW * _L)

    hin = {0: pltpu.async_copy(x_hbm.at[row0], rbufs[0], sem_in)}

    def z_body(i, _):
        for u in range(_UN):
            zbuf[pl.ds((i * _UN + u) * _L, _L)] = zero
        return 0
    lax.fori_loop(0, _N // _L // _UN, z_body, 0)

    def zp_body(i, _):
        pairbuf[pl.ds(i * _L, _L)] = izero
        return 0
    lax.fori_loop(0, _PV, zp_body, 0)

    hout = None
    prev_n = None

    for r in range(rpw):
        row = row0 + r
        hin[r].wait()
        if r + 1 < rpw:
            hin[r + 1] = pltpu.async_copy(
                x_hbm.at[row + 1], rbufs[(r + 1) % 2], sem_in)
        rbuf = rbufs[r % 2]

        def max_body(g, acc):
            t = [rbuf[pl.ds((g * _GW + j) * _L, _L)] for j in range(_GW)]
            while len(t) > 1:
                t = [jnp.maximum(t[i], t[i + 1]) for i in range(0, len(t), 2)]
            gbuf[pl.ds(g * _L, _L)] = t[0]
            return jnp.maximum(acc, t[0])

        acc = lax.fori_loop(0, _NG, max_body, neg)
        m = jnp.max(acc)
        thr = jnp.broadcast_to(m - 1.0, (_L,))

        if hout is not None:
            hout.wait()

            def rst_body(i, _):
                for u in range(_UN2):
                    p = plsc.load_gather(c2pos, [(i * _UN2 + u) * _L + lane])
                    plsc.store_scatter(zbuf, [p], zero)
                return 0
            lax.fori_loop(0, prev_n, rst_body, 0)

        def pair_body(i, cur):
            for u in range(_UN2):
                g = i * _UN2 + u
                gm = gbuf[pl.ds(g * _L, _L)]
                keep = gm > thr
                base = jnp.broadcast_to(g * (_GW * _L), (_L,)) + lane
                plsc.store_scatter(pairbuf, [cur], base, mask=keep)
                cur = cur + jnp.where(keep, _L, 0)
            return cur

        pcur = lax.fori_loop(0, _NG // _UN2, pair_body, lane)
        pcnt = lax.shift_right_logical(pcur - lane, 4)
        kmax = jnp.max(pcnt)

        def scan_body(k, c):
            cur, cnt = c
            b = plsc.load_gather(pairbuf, [k * _L + lane])
            vp = k < pcnt
            for j in range(_GW):
                idx = b + j * _L
                v = plsc.load_gather(rbuf, [idx])
                keep = vp & (v > thr)
                plsc.store_scatter(c2vals, [cur], v, mask=keep)
                plsc.store_scatter(c2pos, [cur], idx, mask=keep)
                cur = cur + jnp.where(keep, _L, 0)
                cnt = cnt + jnp.where(keep, 1, 0)
            return cur, cnt

        _, cnt2 = lax.fori_loop(0, kmax, scan_body, (lane, izero))
        nch2 = jnp.max(cnt2)
        n_new = (nch2 + (_UN2 - 1)) // _UN2

        def fill_body(j, _):
            idxf = jnp.minimum(j, _CAP2 - 1) * _L + lane
            plsc.store_scatter(c2vals, [idxf], sent, mask=j >= cnt2)
            plsc.store_scatter(c2pos, [idxf], izero, mask=j >= cnt2)
            return 0
        lax.fori_loop(0, n_new * _UN2, fill_body, 0)

        def newton(_, t):
            def ch(i, acc2):
                sacc, kacc = acc2
                for u in range(_UN2):
                    v = plsc.load_gather(c2vals, [(i * _UN2 + u) * _L + lane])
                    act = v > t
                    sacc = sacc + jnp.where(act, v, 0.0)
                    kacc = kacc + jnp.where(act, 1.0, 0.0)
                return sacc, kacc
            sacc, kacc = lax.fori_loop(0, n_new, ch, (zero, zero))
            sv = jnp.broadcast_to(jnp.sum(sacc) - 1.0, (_L,))
            kv = jnp.broadcast_to(jnp.sum(kacc), (_L,))
            return sv / kv

        tau = lax.fori_loop(0, _NEWTON_ITERS, newton,
                            jnp.broadcast_to(m - 1.0, (_L,)))

        def sc_out(i, _):
            for u in range(_UN2):
                j = i * _UN2 + u
                v = plsc.load_gather(c2vals, [j * _L + lane])
                p = plsc.load_gather(c2pos, [j * _L + lane])
                plsc.store_scatter(zbuf, [p], jnp.maximum(v - tau, 0.0),
                                   mask=v > thr)
            return 0
        lax.fori_loop(0, n_new, sc_out, 0)

        hout = pltpu.async_copy(zbuf, o_hbm.at[row], sem_out)
        prev_n = n_new

    hout.wait()


@jax.jit
def kernel(input_tensor):
    mesh = plsc.VectorSubcoreMesh(core_axis_name="c", subcore_axis_name="s")
    return pl.kernel(
        _sc_body,
        out_type=jax.ShapeDtypeStruct((_ROWS, _N), jnp.float32),
        mesh=mesh,
        scratch_types=[
            pltpu.VMEM((_N,), jnp.float32),
            pltpu.VMEM((_N,), jnp.float32),
            pltpu.VMEM((_N,), jnp.float32),
            pltpu.VMEM((_N // _GW,), jnp.float32),
            pltpu.VMEM((_L * _PV,), jnp.int32),
            pltpu.VMEM((_L * _CAP2,), jnp.float32),
            pltpu.VMEM((_L * _CAP2,), jnp.int32),
            pltpu.SemaphoreType.DMA,
            pltpu.SemaphoreType.DMA,
        ],
        compiler_params=pltpu.CompilerParams(needs_layout_passes=False),
    )(input_tensor)


# speedup vs baseline: 2.5869x; 1.0228x over previous
import jax
import jax.numpy as jnp
from jax import lax
from jax.experimental import pallas as pl
from jax.experimental.pallas import tpu as pltpu
from jax.experimental.pallas import tpu_sc as plsc

_L = 16
_ROWS, _N = 128, 32768
_UN = 8
_UN2 = 4
_GW = 16
_PV = 256
_CAP2 = 256
_NEWTON_ITERS = 12
_NEG = -3.0e38


def _sc_body(x_hbm, o_hbm, rb0, rb1, zbuf, gbuf, pairbuf, c2vals, c2pos,
             sem_in, sem_out):
    info = plsc.get_sparse_core_info()
    nc, ns = info.num_cores, info.num_subcores
    rpw = _ROWS // (nc * ns)
    wid = lax.axis_index("s") * nc + lax.axis_index("c")
    lane = lax.iota(jnp.int32, _L)
    zero = jnp.zeros((_L,), jnp.float32)
    izero = jnp.zeros((_L,), jnp.int32)
    sent = jnp.full((_L,), _NEG, jnp.float32)
    neg = jnp.full((_L,), _NEG, jnp.float32)
    rbufs = [rb0, rb1]
    row0 = wid * rpw
    _NG = _N // (_GW * _L)

    hin = {0: pltpu.async_copy(x_hbm.at[row0], rbufs[0], sem_in)}

    def z_body(i, _):
        for u in range(_UN):
            zbuf[pl.ds((i * _UN + u) * _L, _L)] = zero
        return 0
    lax.fori_loop(0, _N // _L // _UN, z_body, 0)

    def zp_body(i, _):
        pairbuf[pl.ds(i * _L, _L)] = izero
        return 0
    lax.fori_loop(0, _PV, zp_body, 0)

    hout = None
    prev_n = None

    for r in range(rpw):
        row = row0 + r
        hin[r].wait()
        if r + 1 < rpw:
            hin[r + 1] = pltpu.async_copy(
                x_hbm.at[row + 1], rbufs[(r + 1) % 2], sem_in)
        rbuf = rbufs[r % 2]

        def max_body(g, acc):
            t = [rbuf[pl.ds((g * _GW + j) * _L, _L)] for j in range(_GW)]
            while len(t) > 1:
                t = [jnp.maximum(t[i], t[i + 1]) for i in range(0, len(t), 2)]
            gbuf[pl.ds(g * _L, _L)] = t[0]
            return jnp.maximum(acc, t[0])

        acc = lax.fori_loop(0, _NG, max_body, neg)
        m = jnp.max(acc)
        thr = jnp.broadcast_to(m - 1.0, (_L,))

        if hout is not None:
            hout.wait()

            def rst_body(i, _):
                for u in range(_UN2):
                    p = c2pos[pl.ds((i * _UN2 + u) * _L, _L)]
                    plsc.store_scatter(zbuf, [p], zero)
                return 0
            lax.fori_loop(0, prev_n, rst_body, 0)

        def pair_body(i, cur):
            for u in range(_UN2):
                g = i * _UN2 + u
                gm = gbuf[pl.ds(g * _L, _L)]
                keep = gm > thr
                base = jnp.broadcast_to(g * (_GW * _L), (_L,)) + lane
                plsc.store_scatter(pairbuf, [cur], base, mask=keep)
                cur = cur + jnp.where(keep, _L, 0)
            return cur

        pcur = lax.fori_loop(0, _NG // _UN2, pair_body, lane)
        pcnt = lax.shift_right_logical(pcur - lane, 4)
        kmax = jnp.max(pcnt)

        def scan_body(k, c):
            cur, cnt = c
            b = pairbuf[pl.ds(k * _L, _L)]
            vp = k < pcnt
            for j in range(_GW):
                idx = b + j * _L
                v = plsc.load_gather(rbuf, [idx])
                keep = vp & (v > thr)
                plsc.store_scatter(c2vals, [cur], v, mask=keep)
                plsc.store_scatter(c2pos, [cur], idx, mask=keep)
                cur = cur + jnp.where(keep, _L, 0)
                cnt = cnt + jnp.where(keep, 1, 0)
            return cur, cnt

        _, cnt2 = lax.fori_loop(0, kmax, scan_body, (lane, izero))
        nch2 = jnp.max(cnt2)
        n_new = (nch2 + (_UN2 - 1)) // _UN2

        def fill_body(j, _):
            off = jnp.minimum(j, _CAP2 - 1) * _L
            mask = j >= cnt2
            c2vals[pl.ds(off, _L)] = jnp.where(
                mask, sent, c2vals[pl.ds(off, _L)])
            c2pos[pl.ds(off, _L)] = jnp.where(
                mask, izero, c2pos[pl.ds(off, _L)])
            return 0
        lax.fori_loop(0, n_new * _UN2, fill_body, 0)

        def newton(_, t):
            def ch(i, acc2):
                sacc, kacc = acc2
                for u in range(_UN2):
                    v = c2vals[pl.ds((i * _UN2 + u) * _L, _L)]
                    act = v > t
                    sacc = sacc + jnp.where(act, v, 0.0)
                    kacc = kacc + jnp.where(act, 1.0, 0.0)
                return sacc, kacc
            sacc, kacc = lax.fori_loop(0, n_new, ch, (zero, zero))
            sv = jnp.broadcast_to(jnp.sum(sacc) - 1.0, (_L,))
            kv = jnp.broadcast_to(jnp.sum(kacc), (_L,))
            return sv / kv

        tau = lax.fori_loop(0, _NEWTON_ITERS, newton,
                            jnp.broadcast_to(m - 1.0, (_L,)))

        def sc_out(i, _):
            for u in range(_UN2):
                j = i * _UN2 + u
                v = c2vals[pl.ds(j * _L, _L)]
                p = c2pos[pl.ds(j * _L, _L)]
                plsc.store_scatter(zbuf, [p], jnp.maximum(v - tau, 0.0),
                                   mask=v > thr)
            return 0
        lax.fori_loop(0, n_new, sc_out, 0)

        hout = pltpu.async_copy(zbuf, o_hbm.at[row], sem_out)
        prev_n = n_new

    hout.wait()


@jax.jit
def kernel(input_tensor):
    mesh = plsc.VectorSubcoreMesh(core_axis_name="c", subcore_axis_name="s")
    return pl.kernel(
        _sc_body,
        out_type=jax.ShapeDtypeStruct((_ROWS, _N), jnp.float32),
        mesh=mesh,
        scratch_types=[
            pltpu.VMEM((_N,), jnp.float32),
            pltpu.VMEM((_N,), jnp.float32),
            pltpu.VMEM((_N,), jnp.float32),
            pltpu.VMEM((_N // _GW,), jnp.float32),
            pltpu.VMEM((_L * _PV,), jnp.int32),
            pltpu.VMEM((_L * _CAP2,), jnp.float32),
            pltpu.VMEM((_L * _CAP2,), jnp.int32),
            pltpu.SemaphoreType.DMA,
            pltpu.SemaphoreType.DMA,
        ],
        compiler_params=pltpu.CompilerParams(needs_layout_passes=False),
    )(input_tensor)
